# Initial kernel scaffold; baseline (speedup 1.0000x reference)
import jax
import jax.numpy as jnp
from jax.experimental import pallas as pl


def kernel(input_ids, table):
    raise NotImplementedError("write your pallas kernel here")


# speedup vs baseline: 1.0436x; 1.0436x over previous
import functools

import jax
import jax.numpy as jnp
from jax.experimental import pallas as pl
from jax.experimental.pallas import tpu as pltpu
from jax.experimental.pallas import tpu_sc as plsc

_WINDOW = 128


def _gather_call(table, idx2d, n_idx, d):
    mesh = plsc.VectorSubcoreMesh(
        core_axis_name="core", subcore_axis_name="subcore"
    )

    @functools.partial(
        pl.kernel,
        out_type=jax.ShapeDtypeStruct((n_idx, d), table.dtype),
        mesh=mesh,
        compiler_params=pltpu.CompilerParams(use_tc_tiling_on_sc=False),
    )
    def gather_kernel(table_hbm, idx_hbm, out_hbm):
        def body(i_vmem, o_vmem):
            pltpu.sync_copy(table_hbm.at[i_vmem.at[0]], o_vmem)

        pltpu.emit_pipeline(
            body,
            grid=(n_idx // _WINDOW,),
            in_specs=[
                pl.BlockSpec((1, _WINDOW), index_map=lambda i: (0, i))
            ],
            out_specs=[
                pl.BlockSpec((_WINDOW, d), index_map=lambda i: (i, 0))
            ],
            core_axis_name=("core", "subcore"),
            dimension_semantics=(pltpu.PARALLEL,),
        )(idx_hbm, out_hbm)

    return gather_kernel(table, idx2d)


def kernel(input_ids, table):
    b, s = input_ids.shape
    v, d = table.shape
    n = b * s
    idx2d = input_ids.reshape(1, n).astype(jnp.int32)
    out = _gather_call(table, idx2d, n, d)
    return out.reshape(b, s, d)


# speedup vs baseline: 1.0993x; 1.0534x over previous
import functools

import jax
import jax.numpy as jnp
from jax.experimental import pallas as pl
from jax.experimental.pallas import tpu as pltpu
from jax.experimental.pallas import tpu_sc as plsc

_WINDOW = 512


def _gather_call(table, idx2d, n_idx, d):
    mesh = plsc.VectorSubcoreMesh(
        core_axis_name="core", subcore_axis_name="subcore"
    )

    @functools.partial(
        pl.kernel,
        out_type=jax.ShapeDtypeStruct((n_idx, d), table.dtype),
        mesh=mesh,
        compiler_params=pltpu.CompilerParams(use_tc_tiling_on_sc=False),
    )
    def gather_kernel(table_hbm, idx_hbm, out_hbm):
        def body(i_vmem, o_vmem):
            pltpu.sync_copy(table_hbm.at[i_vmem.at[0]], o_vmem)

        pltpu.emit_pipeline(
            body,
            grid=(n_idx // _WINDOW,),
            in_specs=[
                pl.BlockSpec((1, _WINDOW), index_map=lambda i: (0, i))
            ],
            out_specs=[
                pl.BlockSpec((_WINDOW, d), index_map=lambda i: (i, 0))
            ],
            core_axis_name=("core", "subcore"),
            dimension_semantics=(pltpu.PARALLEL,),
        )(idx_hbm, out_hbm)

    return gather_kernel(table, idx2d)


def kernel(input_ids, table):
    b, s = input_ids.shape
    v, d = table.shape
    n = b * s
    idx2d = input_ids.reshape(1, n).astype(jnp.int32)
    out = _gather_call(table, idx2d, n, d)
    return out.reshape(b, s, d)


# speedup vs baseline: 1.3358x; 1.2151x over previous
import functools

import jax
import jax.numpy as jnp
from jax import lax
from jax.experimental import pallas as pl
from jax.experimental.pallas import tpu as pltpu
from jax.experimental.pallas import tpu_sc as plsc

_BCHUNK = 1024
_L = 16


def _gather_call(table, ids_t, b, s, d):
    nbb = b // 128
    ntr = d // 8
    nsteps = b // _BCHUNK
    bbpc = _BCHUNK // 128
    mesh = plsc.VectorSubcoreMesh(
        core_axis_name="core", subcore_axis_name="subcore"
    )

    @functools.partial(
        pl.kernel,
        out_type=jax.ShapeDtypeStruct((s, ntr, nbb, 8, 128), table.dtype),
        mesh=mesh,
        compiler_params=pltpu.CompilerParams(
            use_tc_tiling_on_sc=False, needs_layout_passes=False
        ),
    )
    def gather_kernel(table_hbm, idx_hbm, out_hbm):
        def body(i_vmem, o_vmem):
            def scoped(rows_v):
                pltpu.sync_copy(table_hbm.at[i_vmem.at[0]], rows_v)
                lane = lax.iota(jnp.int32, _L)

                @pl.loop(0, ntr)
                def _(tr):
                    @pl.loop(0, bbpc)
                    def _(bbl):
                        @pl.loop(0, 8)
                        def _(c8):
                            col = jnp.full((_L,), 0, jnp.int32) + tr * 8 + c8
                            base = bbl * 128
                            for j in range(128 // _L):
                                ridx = base + j * _L + lane
                                vals = plsc.load_gather(rows_v, [ridx, col])
                                o_vmem[0, tr, bbl, c8, pl.ds(j * _L, _L)] = (
                                    vals
                                )

            pl.run_scoped(
                scoped, pltpu.VMEM((_BCHUNK, d), jnp.float32)
            )

        pltpu.emit_pipeline(
            body,
            grid=(s, nsteps),
            in_specs=[
                pl.BlockSpec((1, _BCHUNK), index_map=lambda si, bi: (si, bi))
            ],
            out_specs=[
                pl.BlockSpec(
                    (1, ntr, bbpc, 8, 128),
                    index_map=lambda si, bi: (si, 0, bi, 0, 0),
                )
            ],
            core_axis_name=("core", "subcore"),
            dimension_semantics=(pltpu.PARALLEL, pltpu.PARALLEL),
        )(idx_hbm, out_hbm)

    return gather_kernel(table, ids_t)


def kernel(input_ids, table):
    b, s = input_ids.shape
    v, d = table.shape
    ids_t = input_ids.T.astype(jnp.int32)
    out5 = _gather_call(table, ids_t, b, s, d)
    return out5.transpose(2, 4, 0, 1, 3).reshape(b, s, d)


# speedup vs baseline: 1.3437x; 1.0059x over previous
import functools

import jax
import jax.numpy as jnp
from jax import lax
from jax.experimental import pallas as pl
from jax.experimental.pallas import tpu as pltpu
from jax.experimental.pallas import tpu_sc as plsc

_BCHUNK = 1024
_L = 16


def _gather_call(table, ids_t, b, s, d):
    nbb = b // 128
    ntr = d // 8
    nsteps = b // _BCHUNK
    bbpc = _BCHUNK // 128
    mesh = plsc.VectorSubcoreMesh(
        core_axis_name="core", subcore_axis_name="subcore"
    )

    @functools.partial(
        pl.kernel,
        out_type=jax.ShapeDtypeStruct((s, ntr, nbb, 8, 128), table.dtype),
        mesh=mesh,
        compiler_params=pltpu.CompilerParams(
            use_tc_tiling_on_sc=False, needs_layout_passes=False
        ),
    )
    def gather_kernel(table_hbm, idx_hbm, out_hbm):
        def body(i_vmem, o_vmem):
            def scoped(rows_v):
                pltpu.sync_copy(table_hbm.at[i_vmem.at[0]], rows_v)
                lane = lax.iota(jnp.int32, _L)

                @pl.loop(0, ntr * bbpc)
                def _(tb):
                    tr = tb // bbpc
                    bbl = tb - tr * bbpc
                    base = bbl * 128
                    for c8 in range(8):
                        col = jnp.full((_L,), 0, jnp.int32) + (tr * 8 + c8)
                        for j in range(128 // _L):
                            ridx = base + j * _L + lane
                            vals = plsc.load_gather(rows_v, [ridx, col])
                            o_vmem[0, tr, bbl, c8, pl.ds(j * _L, _L)] = vals

            pl.run_scoped(
                scoped, pltpu.VMEM((_BCHUNK, d), jnp.float32)
            )

        pltpu.emit_pipeline(
            body,
            grid=(s, nsteps),
            in_specs=[
                pl.BlockSpec((1, _BCHUNK), index_map=lambda si, bi: (si, bi))
            ],
            out_specs=[
                pl.BlockSpec(
                    (1, ntr, bbpc, 8, 128),
                    index_map=lambda si, bi: (si, 0, bi, 0, 0),
                )
            ],
            core_axis_name=("core", "subcore"),
            dimension_semantics=(pltpu.PARALLEL, pltpu.PARALLEL),
        )(idx_hbm, out_hbm)

    return gather_kernel(table, ids_t)


def kernel(input_ids, table):
    b, s = input_ids.shape
    v, d = table.shape
    ids_t = input_ids.T.astype(jnp.int32)
    out5 = _gather_call(table, ids_t, b, s, d)
    return out5.transpose(2, 4, 0, 1, 3).reshape(b, s, d)


# speedup vs baseline: 2.0142x; 1.4990x over previous
import functools

import jax
import jax.numpy as jnp
from jax import lax
from jax.experimental import pallas as pl
from jax.experimental.pallas import tpu as pltpu
from jax.experimental.pallas import tpu_sc as plsc

_BCHUNK = 1024
_L = 16


def _gather_call(table, ids_t, b, s, d):
    nbb = b // 128
    ntr = d // 8
    nsteps = b // _BCHUNK
    bbpc = _BCHUNK // 128
    mesh = plsc.VectorSubcoreMesh(
        core_axis_name="core", subcore_axis_name="subcore"
    )

    @functools.partial(
        pl.kernel,
        out_type=jax.ShapeDtypeStruct((s, ntr, nbb, 8, 128), table.dtype),
        mesh=mesh,
        compiler_params=pltpu.CompilerParams(
            use_tc_tiling_on_sc=False, needs_layout_passes=False
        ),
    )
    def gather_kernel(table_hbm, idx_hbm, out_hbm):
        def body(i_vmem, o_vmem):
            def scoped(rows_v):
                pltpu.sync_copy(table_hbm.at[i_vmem.at[0]], rows_v)
                lane = lax.iota(jnp.int32, _L)
                zero = jnp.zeros((_L,), jnp.int32)

                @pl.loop(0, d)
                def _(c0):
                    t = (c0 + lane) & (d - 1)
                    tr_v = t >> 3
                    c8_v = t & 7
                    for bbl in range(bbpc):
                        bblv = zero + bbl
                        for rg in range(128 // _L):
                            b1_v = rg * _L + lane
                            r_loc = bbl * 128 + b1_v
                            vals = plsc.load_gather(rows_v, [r_loc, t])
                            plsc.store_scatter(
                                o_vmem,
                                [zero, tr_v, bblv, c8_v, b1_v],
                                vals,
                            )

            pl.run_scoped(
                scoped, pltpu.VMEM((_BCHUNK, d), jnp.float32)
            )

        pltpu.emit_pipeline(
            body,
            grid=(s, nsteps),
            in_specs=[
                pl.BlockSpec((1, _BCHUNK), index_map=lambda si, bi: (si, bi))
            ],
            out_specs=[
                pl.BlockSpec(
                    (1, ntr, bbpc, 8, 128),
                    index_map=lambda si, bi: (si, 0, bi, 0, 0),
                )
            ],
            core_axis_name=("core", "subcore"),
            dimension_semantics=(pltpu.PARALLEL, pltpu.PARALLEL),
        )(idx_hbm, out_hbm)

    return gather_kernel(table, ids_t)


def kernel(input_ids, table):
    b, s = input_ids.shape
    v, d = table.shape
    ids_t = input_ids.T.astype(jnp.int32)
    out5 = _gather_call(table, ids_t, b, s, d)
    return out5.transpose(2, 4, 0, 1, 3).reshape(b, s, d)


# speedup vs baseline: 2.6942x; 1.3376x over previous
import functools

import jax
import jax.numpy as jnp
from jax import lax
from jax.experimental import pallas as pl
from jax.experimental.pallas import tpu as pltpu
from jax.experimental.pallas import tpu_sc as plsc

_BCHUNK = 1024
_L = 16


def _gather_call(table, ids_t, b, s, d):
    nbb = b // 128
    ntr = d // 8
    nsteps = b // _BCHUNK
    bbpc = _BCHUNK // 128
    mesh = plsc.VectorSubcoreMesh(
        core_axis_name="core", subcore_axis_name="subcore"
    )

    @functools.partial(
        pl.kernel,
        out_type=jax.ShapeDtypeStruct((s, ntr, nbb, 8, 128), table.dtype),
        mesh=mesh,
        compiler_params=pltpu.CompilerParams(
            use_tc_tiling_on_sc=False, needs_layout_passes=False
        ),
    )
    def gather_kernel(table_hbm, idx_hbm, out_hbm):
        def body(i_vmem, o_vmem):
            def scoped(rows_v):
                pltpu.sync_copy(table_hbm.at[i_vmem.at[0]], rows_v)
                lane = lax.iota(jnp.int32, _L)
                zero = jnp.zeros((_L,), jnp.int32)

                @pl.loop(0, d)
                def _(c0):
                    t = (c0 + lane) & (d - 1)
                    tr_v = t >> 3
                    c8_v = t & 7
                    for bbl in range(bbpc):
                        bblv = zero + bbl
                        for rg in range(128 // _L):
                            b1_v = rg * _L + lane
                            r_loc = bbl * 128 + b1_v
                            vals = plsc.load_gather(rows_v, [r_loc, t])
                            plsc.store_scatter(
                                o_vmem,
                                [zero, tr_v, bblv, c8_v, b1_v],
                                vals,
                            )

            pl.run_scoped(
                scoped, pltpu.VMEM((_BCHUNK, d), jnp.float32)
            )

        pltpu.emit_pipeline(
            body,
            grid=(s, nsteps),
            in_specs=[
                pl.BlockSpec((1, _BCHUNK), index_map=lambda si, bi: (si, bi))
            ],
            out_specs=[
                pl.BlockSpec(
                    (1, ntr, bbpc, 8, 128),
                    index_map=lambda si, bi: (si, 0, bi, 0, 0),
                )
            ],
            core_axis_name=("core", "subcore"),
            dimension_semantics=(pltpu.PARALLEL, pltpu.PARALLEL),
        )(idx_hbm, out_hbm)

    return gather_kernel(table, ids_t)


def _detile_call(table_t, v, d):
    nfull = v // 128
    tail = v - nfull * 128
    mesh = plsc.VectorSubcoreMesh(
        core_axis_name="core", subcore_axis_name="subcore"
    )

    @functools.partial(
        pl.kernel,
        out_type=jax.ShapeDtypeStruct((v * d,), table_t.dtype),
        mesh=mesh,
        compiler_params=pltpu.CompilerParams(
            use_tc_tiling_on_sc=True, needs_layout_passes=False
        ),
    )
    def detile_kernel(tab_hbm, out_hbm):
        lane = lax.iota(jnp.int32, _L)

        def transpose_block(blk, half, o_v, ngrp):
            @pl.loop(0, _L)
            def _(c0):
                t16 = (c0 + lane) & (_L - 1)
                cidx = half * _L + t16
                for rg in range(ngrp):
                    r_l = rg * _L + lane
                    vals = plsc.load_gather(blk, [t16, r_l])
                    plsc.store_scatter(o_v, [r_l * d + cidx], vals)

        def body(lo_v, hi_v, o_v):
            transpose_block(lo_v, 0, o_v, 128 // _L)
            transpose_block(hi_v, 1, o_v, 128 // _L)

        pltpu.emit_pipeline(
            body,
            grid=(nfull,),
            in_specs=[
                pl.BlockSpec((_L, 128), index_map=lambda i: (0, i)),
                pl.BlockSpec((_L, 128), index_map=lambda i: (1, i)),
            ],
            out_specs=[
                pl.BlockSpec((128 * d,), index_map=lambda i: (i,))
            ],
            core_axis_name=("core", "subcore"),
            dimension_semantics=(pltpu.PARALLEL,),
        )(tab_hbm, tab_hbm, out_hbm)

        if tail:
            wid = lax.axis_index("subcore") * 2 + lax.axis_index("core")

            @pl.when(wid == 0)
            def _():
                def scoped(lo_v, hi_v, o_v, sem):
                    pltpu.async_copy(
                        tab_hbm.at[pl.ds(0, _L), pl.ds(nfull * 128, tail)],
                        lo_v,
                        sem,
                    ).wait()
                    pltpu.async_copy(
                        tab_hbm.at[pl.ds(_L, _L), pl.ds(nfull * 128, tail)],
                        hi_v,
                        sem,
                    ).wait()
                    transpose_block(lo_v, 0, o_v, tail // _L)
                    transpose_block(hi_v, 1, o_v, tail // _L)
                    pltpu.async_copy(
                        o_v,
                        out_hbm.at[pl.ds(nfull * 128 * d, tail * d)],
                        sem,
                    ).wait()

                pl.run_scoped(
                    scoped,
                    pltpu.VMEM((_L, tail), jnp.float32),
                    pltpu.VMEM((_L, tail), jnp.float32),
                    pltpu.VMEM((tail * d,), jnp.float32),
                    pltpu.SemaphoreType.DMA,
                )

    return detile_kernel(table_t)


def kernel(input_ids, table):
    b, s = input_ids.shape
    v, d = table.shape
    ids_t = input_ids.T.astype(jnp.int32)
    table_rm = _detile_call(table.T, v, d).reshape(v, d)
    out5 = _gather_call(table_rm, ids_t, b, s, d)
    return out5.transpose(2, 4, 0, 1, 3).reshape(b, s, d)


# speedup vs baseline: 3.1319x; 1.1625x over previous
import functools

import jax
import jax.numpy as jnp
from jax import lax
from jax.experimental import pallas as pl
from jax.experimental.pallas import tpu as pltpu
from jax.experimental.pallas import tpu_sc as plsc

_BCHUNK = 1024
_L = 16


def _gather_call(table, ids_t, b, s, d):
    nbb = b // 128
    ntr = d // 8
    nsteps = b // _BCHUNK
    bbpc = _BCHUNK // 128
    mesh = plsc.VectorSubcoreMesh(
        core_axis_name="core", subcore_axis_name="subcore"
    )

    nq = 4
    qrows = _BCHUNK // nq

    @functools.partial(
        pl.kernel,
        out_type=jax.ShapeDtypeStruct((s, ntr, nbb * 1024), table.dtype),
        mesh=mesh,
        compiler_params=pltpu.CompilerParams(
            use_tc_tiling_on_sc=False, needs_layout_passes=False
        ),
    )
    def gather_kernel(table_hbm, idx_hbm, out_hbm):
        def body(i_vmem, o_vmem):
            def scoped(rows_v, sem):
                for q in range(nq):
                    pltpu.async_copy(
                        table_hbm.at[i_vmem.at[0, pl.ds(q * qrows, qrows)]],
                        rows_v.at[pl.ds(q * qrows, qrows)],
                        sem,
                    )
                lane = lax.iota(jnp.int32, _L)
                zero = jnp.zeros((_L,), jnp.int32)

                for q in range(nq):
                    pltpu.make_async_copy(
                        table_hbm.at[
                            i_vmem.at[0, pl.ds(q * qrows, qrows)]
                        ],
                        rows_v.at[pl.ds(q * qrows, qrows)],
                        sem,
                    ).wait()

                    @pl.loop(0, d)
                    def _(c0):
                        t = (c0 + lane) & (d - 1)
                        tr_v = t >> 3
                        cl = ((t & 7) << 7) + lane
                        for bbl in range(
                            q * bbpc // nq, (q + 1) * bbpc // nq
                        ):
                            for rg in range(128 // _L):
                                r_loc = bbl * 128 + rg * _L + lane
                                flat_v = cl + (bbl * 1024 + rg * _L)
                                vals = plsc.load_gather(
                                    rows_v, [r_loc, t]
                                )
                                plsc.store_scatter(
                                    o_vmem,
                                    [zero, tr_v, flat_v],
                                    vals,
                                )

            pl.run_scoped(
                scoped,
                pltpu.VMEM((_BCHUNK, d), jnp.float32),
                pltpu.SemaphoreType.DMA,
            )

        pltpu.emit_pipeline(
            body,
            grid=(s, nsteps),
            in_specs=[
                pl.BlockSpec((1, _BCHUNK), index_map=lambda si, bi: (si, bi))
            ],
            out_specs=[
                pl.BlockSpec(
                    (1, ntr, bbpc * 1024),
                    index_map=lambda si, bi: (si, 0, bi),
                )
            ],
            core_axis_name=("core", "subcore"),
            dimension_semantics=(pltpu.PARALLEL, pltpu.PARALLEL),
        )(idx_hbm, out_hbm)

    return gather_kernel(table, ids_t)


def _detile_call(table_t, v, d):
    nfull = v // 128
    tail = v - nfull * 128
    mesh = plsc.VectorSubcoreMesh(
        core_axis_name="core", subcore_axis_name="subcore"
    )

    @functools.partial(
        pl.kernel,
        out_type=jax.ShapeDtypeStruct((v * d,), table_t.dtype),
        mesh=mesh,
        compiler_params=pltpu.CompilerParams(
            use_tc_tiling_on_sc=True, needs_layout_passes=False
        ),
    )
    def detile_kernel(tab_hbm, out_hbm):
        lane = lax.iota(jnp.int32, _L)

        def transpose_block(blk, half, o_v, ngrp):
            @pl.loop(0, _L)
            def _(c0):
                t16 = (c0 + lane) & (_L - 1)
                cidx = half * _L + t16
                for rg in range(ngrp):
                    r_l = rg * _L + lane
                    vals = plsc.load_gather(blk, [t16, r_l])
                    plsc.store_scatter(o_v, [r_l * d + cidx], vals)

        def body(lo_v, hi_v, o_v):
            transpose_block(lo_v, 0, o_v, 128 // _L)
            transpose_block(hi_v, 1, o_v, 128 // _L)

        pltpu.emit_pipeline(
            body,
            grid=(nfull,),
            in_specs=[
                pl.BlockSpec((_L, 128), index_map=lambda i: (0, i)),
                pl.BlockSpec((_L, 128), index_map=lambda i: (1, i)),
            ],
            out_specs=[
                pl.BlockSpec((128 * d,), index_map=lambda i: (i,))
            ],
            core_axis_name=("core", "subcore"),
            dimension_semantics=(pltpu.PARALLEL,),
        )(tab_hbm, tab_hbm, out_hbm)

        if tail:
            wid = lax.axis_index("subcore") * 2 + lax.axis_index("core")

            @pl.when(wid == 0)
            def _():
                def scoped(lo_v, hi_v, o_v, sem):
                    pltpu.async_copy(
                        tab_hbm.at[pl.ds(0, _L), pl.ds(nfull * 128, tail)],
                        lo_v,
                        sem,
                    ).wait()
                    pltpu.async_copy(
                        tab_hbm.at[pl.ds(_L, _L), pl.ds(nfull * 128, tail)],
                        hi_v,
                        sem,
                    ).wait()
                    transpose_block(lo_v, 0, o_v, tail // _L)
                    transpose_block(hi_v, 1, o_v, tail // _L)
                    pltpu.async_copy(
                        o_v,
                        out_hbm.at[pl.ds(nfull * 128 * d, tail * d)],
                        sem,
                    ).wait()

                pl.run_scoped(
                    scoped,
                    pltpu.VMEM((_L, tail), jnp.float32),
                    pltpu.VMEM((_L, tail), jnp.float32),
                    pltpu.VMEM((tail * d,), jnp.float32),
                    pltpu.SemaphoreType.DMA,
                )

    return detile_kernel(table_t)


def kernel(input_ids, table):
    b, s = input_ids.shape
    v, d = table.shape
    ids_t = input_ids.T.astype(jnp.int32)
    table_rm = _detile_call(table.T, v, d).reshape(v, d)
    out3 = _gather_call(table_rm, ids_t, b, s, d)
    out5 = out3.reshape(s, d // 8, b // 128, 8, 128)
    return out5.transpose(2, 4, 0, 1, 3).reshape(b, s, d)


# speedup vs baseline: 3.1332x; 1.0004x over previous
import functools

import jax
import jax.numpy as jnp
from jax import lax
from jax.experimental import pallas as pl
from jax.experimental.pallas import tpu as pltpu
from jax.experimental.pallas import tpu_sc as plsc

_BCHUNK = 1024
_L = 16


def _gather_call(table, ids_t, b, s, d):
    nbb = b // 128
    ntr = d // 8
    nsteps = b // _BCHUNK
    bbpc = _BCHUNK // 128
    mesh = plsc.VectorSubcoreMesh(
        core_axis_name="core", subcore_axis_name="subcore"
    )

    nq = 4
    qrows = _BCHUNK // nq

    @functools.partial(
        pl.kernel,
        out_type=jax.ShapeDtypeStruct((s, ntr, nbb * 1024), table.dtype),
        mesh=mesh,
        compiler_params=pltpu.CompilerParams(
            use_tc_tiling_on_sc=False, needs_layout_passes=False
        ),
    )
    def gather_kernel(table_hbm, idx_hbm, out_hbm):
        def body(i_vmem, o_vmem):
            def scoped(rows_v, sem):
                for q in range(nq):
                    pltpu.async_copy(
                        table_hbm.at[i_vmem.at[0, pl.ds(q * qrows, qrows)]],
                        rows_v.at[pl.ds(q * qrows, qrows)],
                        sem,
                    )
                lane = lax.iota(jnp.int32, _L)
                zero = jnp.zeros((_L,), jnp.int32)

                for q in range(nq):
                    pltpu.make_async_copy(
                        table_hbm.at[
                            i_vmem.at[0, pl.ds(q * qrows, qrows)]
                        ],
                        rows_v.at[pl.ds(q * qrows, qrows)],
                        sem,
                    ).wait()

                    @pl.loop(0, d)
                    def _(c0):
                        t = (c0 + lane) & (d - 1)
                        tr_v = t >> 3
                        cl = ((t & 7) << 7) + lane
                        for bbl in range(
                            q * bbpc // nq, (q + 1) * bbpc // nq
                        ):
                            for rg in range(128 // _L):
                                r_loc = bbl * 128 + rg * _L + lane
                                flat_v = cl + (bbl * 1024 + rg * _L)
                                vals = plsc.load_gather(
                                    rows_v, [r_loc, t]
                                )
                                plsc.store_scatter(
                                    o_vmem,
                                    [zero, tr_v, flat_v],
                                    vals,
                                )

            pl.run_scoped(
                scoped,
                pltpu.VMEM((_BCHUNK, d), jnp.float32),
                pltpu.SemaphoreType.DMA,
            )

        pltpu.emit_pipeline(
            body,
            grid=(s, nsteps),
            in_specs=[
                pl.BlockSpec((1, _BCHUNK), index_map=lambda si, bi: (si, bi))
            ],
            out_specs=[
                pl.BlockSpec(
                    (1, ntr, bbpc * 1024),
                    index_map=lambda si, bi: (si, 0, bi),
                )
            ],
            core_axis_name=("core", "subcore"),
            dimension_semantics=(pltpu.PARALLEL, pltpu.PARALLEL),
        )(idx_hbm, out_hbm)

    return gather_kernel(table, ids_t)


def _detile_call(table_t, v, d):
    nfull = v // 128
    tail = v - nfull * 128
    mesh = plsc.VectorSubcoreMesh(
        core_axis_name="core", subcore_axis_name="subcore"
    )

    @functools.partial(
        pl.kernel,
        out_type=jax.ShapeDtypeStruct((v * d,), table_t.dtype),
        mesh=mesh,
        compiler_params=pltpu.CompilerParams(
            use_tc_tiling_on_sc=True, needs_layout_passes=False
        ),
    )
    def detile_kernel(tab_hbm, out_hbm):
        lane = lax.iota(jnp.int32, _L)

        def transpose_block(blk, half, o_v, ngrp):
            @pl.loop(0, _L)
            def _(c0):
                t16 = (c0 + lane) & (_L - 1)
                lc = lane * d + (half * _L) + t16
                for rg in range(ngrp):
                    vals = plsc.load_gather(blk, [t16, rg * _L + lane])
                    plsc.store_scatter(o_v, [lc + rg * _L * d], vals)

        def body(lo_v, hi_v, o_v):
            transpose_block(lo_v, 0, o_v, 128 // _L)
            transpose_block(hi_v, 1, o_v, 128 // _L)

        pltpu.emit_pipeline(
            body,
            grid=(nfull,),
            in_specs=[
                pl.BlockSpec((_L, 128), index_map=lambda i: (0, i)),
                pl.BlockSpec((_L, 128), index_map=lambda i: (1, i)),
            ],
            out_specs=[
                pl.BlockSpec((128 * d,), index_map=lambda i: (i,))
            ],
            core_axis_name=("core", "subcore"),
            dimension_semantics=(pltpu.PARALLEL,),
        )(tab_hbm, tab_hbm, out_hbm)

        if tail:
            wid = lax.axis_index("subcore") * 2 + lax.axis_index("core")

            @pl.when(wid == 0)
            def _():
                def scoped(lo_v, hi_v, o_v, sem):
                    pltpu.async_copy(
                        tab_hbm.at[pl.ds(0, _L), pl.ds(nfull * 128, tail)],
                        lo_v,
                        sem,
                    ).wait()
                    pltpu.async_copy(
                        tab_hbm.at[pl.ds(_L, _L), pl.ds(nfull * 128, tail)],
                        hi_v,
                        sem,
                    ).wait()
                    transpose_block(lo_v, 0, o_v, tail // _L)
                    transpose_block(hi_v, 1, o_v, tail // _L)
                    pltpu.async_copy(
                        o_v,
                        out_hbm.at[pl.ds(nfull * 128 * d, tail * d)],
                        sem,
                    ).wait()

                pl.run_scoped(
                    scoped,
                    pltpu.VMEM((_L, tail), jnp.float32),
                    pltpu.VMEM((_L, tail), jnp.float32),
                    pltpu.VMEM((tail * d,), jnp.float32),
                    pltpu.SemaphoreType.DMA,
                )

    return detile_kernel(table_t)


def kernel(input_ids, table):
    b, s = input_ids.shape
    v, d = table.shape
    ids_t = input_ids.T.astype(jnp.int32)
    table_rm = _detile_call(table.T, v, d).reshape(v, d)
    out3 = _gather_call(table_rm, ids_t, b, s, d)
    out5 = out3.reshape(s, d // 8, b // 128, 8, 128)
    return out5.transpose(2, 4, 0, 1, 3).reshape(b, s, d)


# speedup vs baseline: 3.1658x; 1.0104x over previous
import functools

import jax
import jax.numpy as jnp
from jax import lax
from jax.experimental import pallas as pl
from jax.experimental.pallas import tpu as pltpu
from jax.experimental.pallas import tpu_sc as plsc

_BCHUNK = 1024
_L = 16


def _gather_call(table, ids_t, b, s, d):
    nbb = b // 128
    ntr = d // 8
    nsteps = b // _BCHUNK
    bbpc = _BCHUNK // 128
    mesh = plsc.VectorSubcoreMesh(
        core_axis_name="core", subcore_axis_name="subcore"
    )

    nq = 4
    qrows = _BCHUNK // nq

    @functools.partial(
        pl.kernel,
        out_type=jax.ShapeDtypeStruct((s, ntr, nbb * 1024), table.dtype),
        mesh=mesh,
        compiler_params=pltpu.CompilerParams(
            use_tc_tiling_on_sc=False, needs_layout_passes=False
        ),
    )
    def gather_kernel(table_hbm, idx_hbm, out_hbm):
        def body(i_vmem, o_vmem):
            def scoped(rows_v, sem):
                for q in range(nq):
                    pltpu.async_copy(
                        table_hbm.at[i_vmem.at[0, pl.ds(q * qrows, qrows)]],
                        rows_v.at[pl.ds(q * qrows, qrows)],
                        sem,
                    )
                lane = lax.iota(jnp.int32, _L)
                zero = jnp.zeros((_L,), jnp.int32)

                for q in range(nq):
                    pltpu.make_async_copy(
                        table_hbm.at[
                            i_vmem.at[0, pl.ds(q * qrows, qrows)]
                        ],
                        rows_v.at[pl.ds(q * qrows, qrows)],
                        sem,
                    ).wait()

                    @pl.loop(0, d)
                    def _(c0):
                        t = (c0 + lane) & (d - 1)
                        tr_v = t >> 3
                        cl = ((t & 7) << 7) + lane
                        for bbl in range(
                            q * bbpc // nq, (q + 1) * bbpc // nq
                        ):
                            for rg in range(128 // _L):
                                r_loc = bbl * 128 + rg * _L + lane
                                flat_v = cl + (bbl * 1024 + rg * _L)
                                vals = plsc.load_gather(
                                    rows_v, [r_loc, t]
                                )
                                plsc.store_scatter(
                                    o_vmem,
                                    [zero, tr_v, flat_v],
                                    vals,
                                )

            pl.run_scoped(
                scoped,
                pltpu.VMEM((_BCHUNK, d), jnp.float32),
                pltpu.SemaphoreType.DMA,
            )

        pltpu.emit_pipeline(
            body,
            grid=(s, nsteps),
            in_specs=[
                pl.BlockSpec((1, _BCHUNK), index_map=lambda si, bi: (si, bi))
            ],
            out_specs=[
                pl.BlockSpec(
                    (1, ntr, bbpc * 1024),
                    index_map=lambda si, bi: (si, 0, bi),
                )
            ],
            core_axis_name=("core", "subcore"),
            dimension_semantics=(pltpu.PARALLEL, pltpu.PARALLEL),
        )(idx_hbm, out_hbm)

    return gather_kernel(table, ids_t)


def _detile_call(table_t, v, d):
    colw = 512
    nfull = v // colw
    tail = v - nfull * colw
    mesh = plsc.VectorSubcoreMesh(
        core_axis_name="core", subcore_axis_name="subcore"
    )

    @functools.partial(
        pl.kernel,
        out_type=jax.ShapeDtypeStruct((v * d,), table_t.dtype),
        mesh=mesh,
        compiler_params=pltpu.CompilerParams(
            use_tc_tiling_on_sc=True, needs_layout_passes=False
        ),
    )
    def detile_kernel(tab_hbm, out_hbm):
        lane = lax.iota(jnp.int32, _L)

        def transpose_block(blk, half, o_v, ngrp):
            @pl.loop(0, _L)
            def _(c0):
                t16 = (c0 + lane) & (_L - 1)
                lc = lane * d + (half * _L) + t16
                for rg in range(ngrp):
                    vals = plsc.load_gather(blk, [t16, rg * _L + lane])
                    plsc.store_scatter(o_v, [lc + rg * _L * d], vals)

        def body(lo_v, hi_v, o_v):
            transpose_block(lo_v, 0, o_v, colw // _L)
            transpose_block(hi_v, 1, o_v, colw // _L)

        pltpu.emit_pipeline(
            body,
            grid=(nfull,),
            in_specs=[
                pl.BlockSpec((_L, colw), index_map=lambda i: (0, i)),
                pl.BlockSpec((_L, colw), index_map=lambda i: (1, i)),
            ],
            out_specs=[
                pl.BlockSpec((colw * d,), index_map=lambda i: (i,))
            ],
            core_axis_name=("core", "subcore"),
            dimension_semantics=(pltpu.PARALLEL,),
        )(tab_hbm, tab_hbm, out_hbm)

        if tail:
            wid = lax.axis_index("subcore") * 2 + lax.axis_index("core")

            @pl.when(wid == 0)
            def _():
                def scoped(lo_v, hi_v, o_v, sem):
                    pltpu.async_copy(
                        tab_hbm.at[pl.ds(0, _L), pl.ds(nfull * colw, tail)],
                        lo_v,
                        sem,
                    ).wait()
                    pltpu.async_copy(
                        tab_hbm.at[pl.ds(_L, _L), pl.ds(nfull * colw, tail)],
                        hi_v,
                        sem,
                    ).wait()
                    transpose_block(lo_v, 0, o_v, tail // _L)
                    transpose_block(hi_v, 1, o_v, tail // _L)
                    pltpu.async_copy(
                        o_v,
                        out_hbm.at[pl.ds(nfull * colw * d, tail * d)],
                        sem,
                    ).wait()

                pl.run_scoped(
                    scoped,
                    pltpu.VMEM((_L, tail), jnp.float32),
                    pltpu.VMEM((_L, tail), jnp.float32),
                    pltpu.VMEM((tail * d,), jnp.float32),
                    pltpu.SemaphoreType.DMA,
                )

    return detile_kernel(table_t)


def kernel(input_ids, table):
    b, s = input_ids.shape
    v, d = table.shape
    ids_t = input_ids.T.astype(jnp.int32)
    table_rm = _detile_call(table.T, v, d).reshape(v, d)
    out3 = _gather_call(table_rm, ids_t, b, s, d)
    out5 = out3.reshape(s, d // 8, b // 128, 8, 128)
    return out5.transpose(2, 4, 0, 1, 3).reshape(b, s, d)
